# Initial kernel scaffold; baseline (speedup 1.0000x reference)
import jax
import jax.numpy as jnp
from jax.experimental import pallas as pl


def kernel(x, edge_index, W1, b1, g1w, g1b, g1a, W2, b2, g2w, g2b, g2a, Wh, bh):
    raise NotImplementedError("write your pallas kernel here")


# speedup vs baseline: 11.5243x; 11.5243x over previous
import functools

import jax
import jax.numpy as jnp
from jax import lax
from jax.experimental import pallas as pl
from jax.experimental.pallas import tpu as pltpu
from jax.experimental.pallas import tpu_sc as plsc

NC = 2
NS = 16
NT = NC * NS
CHUNK = 128
DEGW = 16


def _sc_mesh():
    return plsc.VectorSubcoreMesh(core_axis_name="c", subcore_axis_name="s")


def _sc_deg(dst3, rpad):
    nch = dst3.shape[1]
    rpt = rpad // NS

    def body(dst_hbm, out_hbm, dstv, buf, acc):
        c = lax.axis_index("c")
        s = lax.axis_index("s")
        w = s * NC + c

        def zb(k, carry):
            buf[k, pl.ds(0, DEGW)] = jnp.zeros((DEGW,), jnp.float32)
            return carry

        lax.fori_loop(0, CHUNK, zb, 0)
        for r in range(rpt // CHUNK):
            pltpu.sync_copy(buf, acc.at[pl.ds(s * rpt + r * CHUNK, CHUNK)])

        def ob(k, carry):
            buf[k, pl.ds(0, DEGW)] = jnp.ones((DEGW,), jnp.float32)
            return carry

        lax.fori_loop(0, CHUNK, ob, 0)
        pltpu.sync_copy(dst_hbm.at[w], dstv)
        plsc.subcore_barrier()

        def step(j, carry):
            pltpu.sync_copy(buf, acc.at[dstv.at[j]], add=True)
            return carry

        lax.fori_loop(0, nch, step, 0)
        plsc.subcore_barrier()
        pltpu.sync_copy(acc.at[pl.ds(s * rpt, rpt)],
                        out_hbm.at[c, pl.ds(s * rpt, rpt)])

    return pl.kernel(
        body,
        out_type=jax.ShapeDtypeStruct((NC, rpad, DEGW), jnp.float32),
        mesh=_sc_mesh(),
        scratch_types=[
            pltpu.VMEM((nch, CHUNK), jnp.int32),
            pltpu.VMEM((CHUNK, DEGW), jnp.float32),
            pltpu.VMEM_SHARED((rpad, DEGW), jnp.float32),
        ],
    )(dst3)


def _sc_edges(h, src3, dst3, rpad):
    nch = src3.shape[1]
    d = h.shape[1]
    rpt = rpad // NS

    def body(h_hbm, src_hbm, dst_hbm, out_hbm, srcv, dstv, rows, acc, sem):
        c = lax.axis_index("c")
        s = lax.axis_index("s")
        w = s * NC + c

        def zb(k, carry):
            rows[k // 8, pl.ds((k % 8) * 16, 16)] = jnp.zeros((16,), jnp.float32)
            return carry

        lax.fori_loop(0, CHUNK * (d // 16), zb, 0)
        for r in range(rpt // CHUNK):
            pltpu.sync_copy(rows, acc.at[pl.ds(s * rpt + r * CHUNK, CHUNK)])
        pltpu.sync_copy(src_hbm.at[w], srcv)
        pltpu.sync_copy(dst_hbm.at[w], dstv)
        plsc.subcore_barrier()

        def step(j, carry):
            pltpu.async_copy(h_hbm.at[srcv.at[j]], rows, sem).wait()
            pltpu.sync_copy(rows, acc.at[dstv.at[j]], add=True)
            return carry

        lax.fori_loop(0, nch, step, 0)
        plsc.subcore_barrier()
        pltpu.sync_copy(acc.at[pl.ds(s * rpt, rpt)],
                        out_hbm.at[c, pl.ds(s * rpt, rpt)])

    return pl.kernel(
        body,
        out_type=jax.ShapeDtypeStruct((NC, rpad, d), jnp.float32),
        mesh=_sc_mesh(),
        scratch_types=[
            pltpu.VMEM((nch, CHUNK), jnp.int32),
            pltpu.VMEM((nch, CHUNK), jnp.int32),
            pltpu.VMEM((CHUNK, d), jnp.float32),
            pltpu.VMEM_SHARED((rpad, d), jnp.float32),
            pltpu.SemaphoreType.DMA,
        ],
    )(h, src3, dst3)


def _tc1(x, W1, degp):
    n = x.shape[0]

    def body(x_ref, w_ref, degp_ref, h_ref, dinv_ref):
        deg = degp_ref[0, :n, 0:1] + degp_ref[1, :n, 0:1] + 1.0
        dinv = lax.rsqrt(deg)
        dinvb = jnp.broadcast_to(dinv, (n, x_ref.shape[1]))
        h = jnp.dot(x_ref[...], w_ref[...], preferred_element_type=jnp.float32)
        h_ref[...] = h * dinvb
        dinv_ref[...] = dinvb

    return pl.pallas_call(
        body,
        out_shape=(
            jax.ShapeDtypeStruct((n, W1.shape[1]), jnp.float32),
            jax.ShapeDtypeStruct((n, W1.shape[1]), jnp.float32),
        ),
    )(x, W1, degp)


def _graphnorm_relu(pre, gw, ga, gb):
    mean = jnp.mean(pre, axis=0, keepdims=True)
    sh = pre - ga * mean
    var = jnp.mean(sh * sh, axis=0, keepdims=True)
    xn = sh * lax.rsqrt(var + 1e-5) * gw + gb
    return jnp.maximum(xn, 0.0)


def _tc2(accp, h1p, dinvb, b1, g1w, g1b, g1a, W2):
    n = h1p.shape[0]

    def body(a_ref, h_ref, d_ref, b_ref, gw_ref, gb_ref, ga_ref, w_ref, o_ref):
        acc = a_ref[0, :n, :] + a_ref[1, :n, :]
        pre = d_ref[...] * (acc + h_ref[...]) + b_ref[...]
        x1 = _graphnorm_relu(pre, gw_ref[...], ga_ref[...], gb_ref[...])
        h2 = jnp.dot(x1, w_ref[...], preferred_element_type=jnp.float32)
        o_ref[...] = d_ref[...] * h2

    return pl.pallas_call(
        body,
        out_shape=jax.ShapeDtypeStruct((n, W2.shape[1]), jnp.float32),
    )(accp, h1p, dinvb, b1, g1w, g1b, g1a, W2)


def _tc3(accp, h2p, dinvb, b2, g2w, g2b, g2a, x, Wh, bh):
    n = h2p.shape[0]
    din = x.shape[1]

    def body(a_ref, h_ref, d_ref, b_ref, gw_ref, gb_ref, ga_ref, x_ref,
             wh_ref, bh_ref, o_ref):
        acc = a_ref[0, :n, :] + a_ref[1, :n, :]
        pre = d_ref[...] * (acc + h_ref[...]) + b_ref[...]
        x2 = _graphnorm_relu(pre, gw_ref[...], ga_ref[...], gb_ref[...])
        out = jnp.dot(x_ref[...], wh_ref[:din, :],
                      preferred_element_type=jnp.float32)
        out = out + jnp.dot(x2, wh_ref[din:, :],
                            preferred_element_type=jnp.float32)
        o_ref[...] = out + bh_ref[...]

    return pl.pallas_call(
        body,
        out_shape=jax.ShapeDtypeStruct((n, Wh.shape[1]), jnp.float32),
    )(accp, h2p, dinvb, b2, g2w, g2b, g2a, x, Wh, bh)


def kernel(x, edge_index, W1, b1, g1w, g1b, g1a, W2, b2, g2w, g2b, g2a, Wh, bh):
    n = x.shape[0]
    e = edge_index.shape[1]
    ei = edge_index.astype(jnp.int32)
    nch = -(-e // (NT * CHUNK))
    e_pad = NT * nch * CHUNK
    rpad = -(-(n + 1) // (NS * CHUNK)) * (NS * CHUNK)
    src3 = jnp.concatenate(
        [ei[0], jnp.zeros((e_pad - e,), jnp.int32)]).reshape(NT, nch, CHUNK)
    dst3 = jnp.concatenate(
        [ei[1], jnp.full((e_pad - e,), n, jnp.int32)]).reshape(NT, nch, CHUNK)

    degp = _sc_deg(dst3, rpad)
    h1p, dinvb = _tc1(x, W1, degp)
    acc1 = _sc_edges(h1p, src3, dst3, rpad)
    h2p = _tc2(acc1, h1p, dinvb, b1, g1w, g1b, g1a, W2)
    acc2 = _sc_edges(h2p, src3, dst3, rpad)
    return _tc3(acc2, h2p, dinvb, b2, g2w, g2b, g2a, x, Wh, bh)
